# baseline (speedup 1.0000x reference)
import jax
import jax.numpy as jnp
from jax import lax
from jax.experimental import pallas as pl
from jax.experimental.pallas import tpu as pltpu
from jax.experimental.pallas import tpu_sc as plsc

_EMBED = 64
_NC = 2
_NS = 16
_NW = _NC * _NS
_L = 16


def _sc_kernel(xs_hbm, tab_hbm, out_hbm, tab_v, idx_v, half0, half1, wsem):
    s_dim, n = xs_hbm.shape
    kt_dim = out_hbm.shape[1]
    it_dim = out_hbm.shape[2]
    nslab = s_dim * kt_dim
    halves = (half0, half1)

    pltpu.sync_copy(tab_hbm, tab_v)

    wid = lax.axis_index("s") * _NC + lax.axis_index("c")
    nloop = (nslab + _NW - 1) // _NW

    def drain(h):
        pltpu.make_async_copy(
            halves[h], out_hbm.at[0, 0, pl.ds(h * it_dim // 2, it_dim // 2)],
            wsem.at[h]).wait()

    def slab_body(m, carry):
        s = wid + _NW * m

        @pl.when(s < nslab)
        def _():
            j = s // kt_dim
            kt = s % kt_dim
            pltpu.sync_copy(xs_hbm.at[j], idx_v)
            ktv = jnp.full((_L,), kt, jnp.int32)
            tabs = [tab_v.at[pl.ds(k8 * 8, 71936)] for k8 in range(kt_dim)]
            for h in range(2):
                buf = halves[h]

                @pl.when(m > 0)
                def _():
                    drain(h)

                @plsc.parallel_loop(0, it_dim // 2, 1, unroll=2)
                def it_body(itl):
                    ibase = (h * (it_dim // 2) + itl) * 2 * _EMBED
                    for b16 in range(8):
                        iv = idx_v[pl.ds(ibase + b16 * _L, _L)] + ktv
                        for k8 in range(kt_dim):
                            val = plsc.load_gather(tabs[k8], [iv])
                            buf[itl, k8, pl.ds(b16 * _L, _L)] = val
                pltpu.async_copy(
                    buf,
                    out_hbm.at[j, kt,
                               pl.ds(h * (it_dim // 2), it_dim // 2)],
                    wsem.at[h])
        return carry

    lax.fori_loop(0, nloop, slab_body, 0)
    drain(0)
    drain(1)


def kernel(x, encoding):
    n, s = x.shape
    v, e = encoding.shape
    kt_dim = e // 8
    it_dim = n // 128
    stride = e + 8
    xs = (x.T * stride).astype(jnp.int32)
    tab = jnp.pad(
        encoding.reshape(v, kt_dim, 8).transpose(0, 2, 1),
        ((0, 0), (0, 1), (0, 0))).reshape(v * stride)
    z = pl.kernel(
        _sc_kernel,
        out_type=jax.ShapeDtypeStruct((s, kt_dim, it_dim, 8, 128),
                                      jnp.float32),
        mesh=plsc.VectorSubcoreMesh(core_axis_name="c", subcore_axis_name="s"),
        compiler_params=pltpu.CompilerParams(
            use_tc_tiling_on_sc=False, needs_layout_passes=False,
            disable_bounds_checks=True),
        scratch_types=[
            pltpu.VMEM((v * stride,), jnp.float32),
            pltpu.VMEM((n,), jnp.int32),
            pltpu.VMEM((it_dim // 2, 8, 128), jnp.float32),
            pltpu.VMEM((it_dim // 2, 8, 128), jnp.float32),
            pltpu.SemaphoreType.DMA((2,)),
        ],
    )(xs, tab)
    return z.transpose(2, 4, 0, 1, 3).reshape(n, s, e)


# speedup vs baseline: 1.2500x; 1.2500x over previous
import jax
import jax.numpy as jnp
from jax import lax
from jax.experimental import pallas as pl
from jax.experimental.pallas import tpu as pltpu
from jax.experimental.pallas import tpu_sc as plsc

_EMBED = 64
_NC = 2
_NS = 16
_NW = _NC * _NS
_L = 16


def _sc_kernel(xs_hbm, tab_hbm, out_hbm, tab_v, idx_v, half0, half1, wsem):
    s_dim, n = xs_hbm.shape
    kt_dim = out_hbm.shape[1]
    it_dim = out_hbm.shape[2]
    nslab = s_dim * kt_dim
    halves = (half0, half1)

    pltpu.sync_copy(tab_hbm, tab_v)

    wid = lax.axis_index("s") * _NC + lax.axis_index("c")
    nloop = (nslab + _NW - 1) // _NW

    def drain(h):
        pltpu.make_async_copy(
            halves[h], out_hbm.at[0, 0, pl.ds(h * it_dim // 2, it_dim // 2)],
            wsem.at[h]).wait()

    def slab_body(m, carry):
        s = wid + _NW * m

        @pl.when(s < nslab)
        def _():
            j = s // kt_dim
            kt = s % kt_dim
            pltpu.sync_copy(xs_hbm.at[j], idx_v)
            rows = [jnp.full((_L,), kt * kt_dim, jnp.int32) + k8
                    for k8 in range(kt_dim)]
            for h in range(2):
                buf = halves[h]

                @pl.when(m > 0)
                def _():
                    drain(h)

                @plsc.parallel_loop(0, it_dim // 2, 1, unroll=2)
                def it_body(itl):
                    ibase = (h * (it_dim // 2) + itl) * 2 * _EMBED
                    for b16 in range(8):
                        iv = idx_v[pl.ds(ibase + b16 * _L, _L)]
                        for k8 in range(kt_dim):
                            val = plsc.load_gather(tab_v, [iv + rows[k8]])
                            buf[itl, k8, pl.ds(b16 * _L, _L)] = val
                pltpu.async_copy(
                    buf,
                    out_hbm.at[j, kt,
                               pl.ds(h * (it_dim // 2), it_dim // 2)],
                    wsem.at[h])
        return carry

    lax.fori_loop(0, nloop, slab_body, 0)
    drain(0)
    drain(1)


def kernel(x, encoding):
    n, s = x.shape
    v, e = encoding.shape
    kt_dim = e // 8
    it_dim = n // 128
    stride = e + 8
    xs = (x.T * stride).astype(jnp.int32)
    tab = jnp.pad(encoding, ((0, 0), (0, 8))).reshape(v * stride)
    z = pl.kernel(
        _sc_kernel,
        out_type=jax.ShapeDtypeStruct((s, kt_dim, it_dim, 8, 128),
                                      jnp.float32),
        mesh=plsc.VectorSubcoreMesh(core_axis_name="c", subcore_axis_name="s"),
        compiler_params=pltpu.CompilerParams(
            use_tc_tiling_on_sc=False, needs_layout_passes=False,
            disable_bounds_checks=True),
        scratch_types=[
            pltpu.VMEM((v * stride,), jnp.float32),
            pltpu.VMEM((n,), jnp.int32),
            pltpu.VMEM((it_dim // 2, 8, 128), jnp.float32),
            pltpu.VMEM((it_dim // 2, 8, 128), jnp.float32),
            pltpu.SemaphoreType.DMA((2,)),
        ],
    )(xs, tab)
    return z.transpose(2, 4, 0, 1, 3).reshape(n, s, e)


# speedup vs baseline: 1.4271x; 1.1417x over previous
import jax
import jax.numpy as jnp
from jax import lax
from jax.experimental import pallas as pl
from jax.experimental.pallas import tpu as pltpu
from jax.experimental.pallas import tpu_sc as plsc

_EMBED = 64
_NC = 2
_NS = 16
_NW = _NC * _NS
_L = 16


def _sc_kernel(xs_hbm, tab_hbm, out_hbm, tab_v, idx_v, half0, half1, wsem):
    s_dim, n = xs_hbm.shape
    kt_dim = out_hbm.shape[1]
    it_dim = out_hbm.shape[2]
    nslab = s_dim * kt_dim
    halves = (half0, half1)

    pltpu.sync_copy(tab_hbm, tab_v)

    wid = lax.axis_index("s") * _NC + lax.axis_index("c")
    nloop = (nslab + _NW - 1) // _NW

    def drain(h):
        pltpu.make_async_copy(
            halves[h], out_hbm.at[0, 0, pl.ds(h * it_dim // 2, it_dim // 2)],
            wsem.at[h]).wait()

    def slab_body(m, carry):
        s = wid + _NW * m

        @pl.when(s < nslab)
        def _():
            j = s // kt_dim
            kt = s % kt_dim
            pltpu.sync_copy(xs_hbm.at[j], idx_v)
            rows = [jnp.full((_L,), kt * kt_dim, jnp.int32) + k8
                    for k8 in range(kt_dim)]
            for h in range(2):
                buf = halves[h]

                @pl.when(m > 0)
                def _():
                    drain(h)

                @plsc.parallel_loop(0, it_dim // 2, 1, unroll=2)
                def it_body(itl):
                    ibase = (h * (it_dim // 2) + itl) * 2 * _EMBED
                    for b16 in range(8):
                        iv = idx_v[pl.ds(ibase + b16 * _L, _L)]
                        for k8 in range(kt_dim):
                            val = plsc.load_gather(tab_v, [iv + rows[k8]])
                            buf[itl, k8, pl.ds(b16 * _L, _L)] = val
                pltpu.async_copy(
                    buf,
                    out_hbm.at[j, kt,
                               pl.ds(h * (it_dim // 2), it_dim // 2)],
                    wsem.at[h])
        return carry

    lax.fori_loop(0, nloop, slab_body, 0)
    drain(0)
    drain(1)


def kernel(x, encoding):
    n, s = x.shape
    v, e = encoding.shape
    kt_dim = e // 8
    it_dim = n // 128
    stride = e + 9
    xs = (x.T * stride).astype(jnp.int32)
    tab = jnp.pad(encoding, ((0, 0), (0, 9))).reshape(v * stride)
    z = pl.kernel(
        _sc_kernel,
        out_type=jax.ShapeDtypeStruct((s, kt_dim, it_dim, 8, 128),
                                      jnp.float32),
        mesh=plsc.VectorSubcoreMesh(core_axis_name="c", subcore_axis_name="s"),
        compiler_params=pltpu.CompilerParams(
            use_tc_tiling_on_sc=False, needs_layout_passes=False,
            disable_bounds_checks=True),
        scratch_types=[
            pltpu.VMEM((v * stride,), jnp.float32),
            pltpu.VMEM((n,), jnp.int32),
            pltpu.VMEM((it_dim // 2, 8, 128), jnp.float32),
            pltpu.VMEM((it_dim // 2, 8, 128), jnp.float32),
            pltpu.SemaphoreType.DMA((2,)),
        ],
    )(xs, tab)
    return z.transpose(2, 4, 0, 1, 3).reshape(n, s, e)


# speedup vs baseline: 1.4394x; 1.0086x over previous
import jax
import jax.numpy as jnp
from jax import lax
from jax.experimental import pallas as pl
from jax.experimental.pallas import tpu as pltpu
from jax.experimental.pallas import tpu_sc as plsc

_EMBED = 64
_NC = 2
_NS = 16
_NW = _NC * _NS
_L = 16


def _sc_kernel(xs_hbm, tab_hbm, out_hbm, tab_v, idx_v, half0, half1, wsem):
    s_dim, n = xs_hbm.shape
    kt_dim = out_hbm.shape[1]
    it_dim = out_hbm.shape[2]
    nslab = s_dim * kt_dim
    halves = (half0, half1)

    pltpu.sync_copy(tab_hbm, tab_v)

    wid = lax.axis_index("s") * _NC + lax.axis_index("c")
    nloop = (nslab + _NW - 1) // _NW

    def drain(h):
        pltpu.make_async_copy(
            halves[h], out_hbm.at[0, 0, pl.ds(h * it_dim // 2, it_dim // 2)],
            wsem.at[h]).wait()

    def slab_body(m, carry):
        s = wid + _NW * m

        @pl.when(s < nslab)
        def _():
            j = s // kt_dim
            kt = s % kt_dim
            pltpu.sync_copy(xs_hbm.at[j], idx_v)
            rows = [jnp.full((_L,), kt * kt_dim, jnp.int32) + k8
                    for k8 in range(kt_dim)]
            for h in range(2):
                buf = halves[h]

                @pl.when(m > 0)
                def _():
                    drain(h)

                @plsc.parallel_loop(0, it_dim // 2, 1, unroll=2)
                def it_body(itl):
                    ibase = (h * (it_dim // 2) + itl) * 2 * _EMBED
                    for b16 in range(8):
                        iv = idx_v[pl.ds(ibase + b16 * _L, _L)]
                        for k8 in range(kt_dim):
                            val = plsc.load_gather(tab_v, [iv + rows[k8]])
                            buf[itl, k8, pl.ds(b16 * _L, _L)] = val
                pltpu.async_copy(
                    buf,
                    out_hbm.at[j, kt,
                               pl.ds(h * (it_dim // 2), it_dim // 2)],
                    wsem.at[h])
        return carry

    lax.fori_loop(0, nloop, slab_body, 0)
    drain(0)
    drain(1)


def kernel(x, encoding):
    n, s = x.shape
    v, e = encoding.shape
    kt_dim = e // 8
    it_dim = n // 128
    stride = e + 1
    xs = (x.T * stride).astype(jnp.int32)
    tab = jnp.pad(encoding, ((0, 0), (0, 1))).reshape(v * stride)
    z = pl.kernel(
        _sc_kernel,
        out_type=jax.ShapeDtypeStruct((s, kt_dim, it_dim, 8, 128),
                                      jnp.float32),
        mesh=plsc.VectorSubcoreMesh(core_axis_name="c", subcore_axis_name="s"),
        compiler_params=pltpu.CompilerParams(
            use_tc_tiling_on_sc=False, needs_layout_passes=False,
            disable_bounds_checks=True),
        scratch_types=[
            pltpu.VMEM((v * stride,), jnp.float32),
            pltpu.VMEM((n,), jnp.int32),
            pltpu.VMEM((it_dim // 2, 8, 128), jnp.float32),
            pltpu.VMEM((it_dim // 2, 8, 128), jnp.float32),
            pltpu.SemaphoreType.DMA((2,)),
        ],
    )(xs, tab)
    return z.transpose(2, 4, 0, 1, 3).reshape(n, s, e)


# speedup vs baseline: 1.5681x; 1.0895x over previous
import jax
import jax.numpy as jnp
from jax import lax
from jax.experimental import pallas as pl
from jax.experimental.pallas import tpu as pltpu
from jax.experimental.pallas import tpu_sc as plsc

_EMBED = 64
_NC = 2
_NS = 16
_NW = _NC * _NS
_L = 16


def _sc_kernel(xs_hbm, tab_hbm, out_hbm, tab_v, idx0, idx1, half0, half1,
               wsem, isem):
    s_dim, n = xs_hbm.shape
    kt_dim = out_hbm.shape[1]
    it_dim = out_hbm.shape[2]
    nslab = s_dim * kt_dim
    halves = (half0, half1)
    idxs = (idx0, idx1)

    pltpu.sync_copy(tab_hbm, tab_v)

    wid = lax.axis_index("s") * _NC + lax.axis_index("c")
    nloop = (nslab + _NW - 1) // _NW

    def drain(h):
        pltpu.make_async_copy(
            halves[h], out_hbm.at[0, 0, pl.ds(h * it_dim // 2, it_dim // 2)],
            wsem.at[h]).wait()

    def prefetch(m, p):
        s = wid + _NW * m

        @pl.when(s < nslab)
        def _():
            pltpu.async_copy(xs_hbm.at[s // kt_dim], idxs[p], isem.at[p])

    prefetch(0, 0)

    def do_slab(m, p):
        s = wid + _NW * m

        @pl.when(s < nslab)
        def _():
            kt = s % kt_dim
            idx_v = idxs[p]
            pltpu.make_async_copy(xs_hbm.at[0], idx_v, isem.at[p]).wait()
            prefetch(m + 1, 1 - p)
            rows = [jnp.full((_L,), kt * kt_dim, jnp.int32) + k8
                    for k8 in range(kt_dim)]
            for h in range(2):
                buf = halves[h]

                @pl.when(m > 0)
                def _():
                    drain(h)

                @plsc.parallel_loop(0, it_dim // 2, 1, unroll=2)
                def it_body(itl):
                    ibase = (h * (it_dim // 2) + itl) * 2 * _EMBED
                    for b16 in range(8):
                        iv = idx_v[pl.ds(ibase + b16 * _L, _L)]
                        for k8 in range(kt_dim):
                            val = plsc.load_gather(tab_v, [iv + rows[k8]])
                            buf[itl, k8, pl.ds(b16 * _L, _L)] = val
                pltpu.async_copy(
                    buf,
                    out_hbm.at[s // kt_dim, kt,
                               pl.ds(h * (it_dim // 2), it_dim // 2)],
                    wsem.at[h])

    def pair_body(m2, carry):
        for p in range(2):
            do_slab(m2 * 2 + p, p)
        return carry

    lax.fori_loop(0, (nloop + 1) // 2, pair_body, 0)
    drain(0)
    drain(1)


def kernel(x, encoding):
    n, s = x.shape
    v, e = encoding.shape
    kt_dim = e // 8
    it_dim = n // 128
    stride = e + 1
    xs = (x.T * stride).astype(jnp.int32)
    tab = jnp.pad(encoding, ((0, 0), (0, 1))).reshape(v * stride)
    z = pl.kernel(
        _sc_kernel,
        out_type=jax.ShapeDtypeStruct((s, kt_dim, it_dim, 8, 128),
                                      jnp.float32),
        mesh=plsc.VectorSubcoreMesh(core_axis_name="c", subcore_axis_name="s"),
        compiler_params=pltpu.CompilerParams(
            use_tc_tiling_on_sc=False, needs_layout_passes=False,
            disable_bounds_checks=True),
        scratch_types=[
            pltpu.VMEM((v * stride,), jnp.float32),
            pltpu.VMEM((n,), jnp.int32),
            pltpu.VMEM((n,), jnp.int32),
            pltpu.VMEM((it_dim // 2, 8, 128), jnp.float32),
            pltpu.VMEM((it_dim // 2, 8, 128), jnp.float32),
            pltpu.SemaphoreType.DMA((2,)),
            pltpu.SemaphoreType.DMA((2,)),
        ],
    )(xs, tab)
    return z.transpose(2, 4, 0, 1, 3).reshape(n, s, e)
